# Initial kernel scaffold; baseline (speedup 1.0000x reference)
import jax
import jax.numpy as jnp
from jax.experimental import pallas as pl


def kernel(fonts, flag_w, contour_w, order_w, arg_w, fc_w, fc_b):
    raise NotImplementedError("write your pallas kernel here")


# speedup vs baseline: 11.6415x; 11.6415x over previous
import functools

import jax
import jax.numpy as jnp
from jax import lax
from jax.experimental import pallas as pl
from jax.experimental.pallas import tpu as pltpu
from jax.experimental.pallas import tpu_sc as plsc

D = 64
LANES = 16
NW = 32
CHUNK = 128
NTAB = 5


def _prep_body(flag_ref, arg_ref, fcw_ref, fcb_ref, flagb_ref, a0_ref, a1_ref):
    fcw = fcw_ref[...]
    flagb_ref[...] = flag_ref[...] + fcb_ref[...]
    a0_ref[...] = lax.dot_general(
        arg_ref[...], fcw[:, :D], (((1,), (1,)), ((), ())),
        preferred_element_type=jnp.float32)
    a1_ref[...] = lax.dot_general(
        arg_ref[...], fcw[:, D:], (((1,), (1,)), ((), ())),
        preferred_element_type=jnp.float32)


def _prep_tables(flag_w, arg_w, fc_w, fc_b):
    n_flag = flag_w.shape[0]
    n_arg = arg_w.shape[0]
    return pl.pallas_call(
        _prep_body,
        out_shape=[
            jax.ShapeDtypeStruct((n_flag, D), jnp.float32),
            jax.ShapeDtypeStruct((n_arg, D), jnp.float32),
            jax.ShapeDtypeStruct((n_arg, D), jnp.float32),
        ],
    )(flag_w, arg_w, fc_w, fc_b.reshape(1, D))


@functools.lru_cache(maxsize=None)
def _make_sc_kernel(n_tokens):
    per_w = n_tokens // NW
    n_chunks = per_w // CHUNK
    assert per_w * NW == n_tokens and n_chunks * CHUNK == per_w
    assert n_chunks % 2 == 0
    mesh = plsc.VectorSubcoreMesh(core_axis_name="c", subcore_axis_name="s")

    @functools.partial(
        pl.kernel,
        mesh=mesh,
        out_type=jax.ShapeDtypeStruct((n_tokens, D), jnp.float32),
        scratch_types=[
            pltpu.VMEM((2, NTAB, CHUNK), jnp.int32),
            pltpu.VMEM((2, NTAB, CHUNK, D), jnp.float32),
            pltpu.SemaphoreType.DMA,
            pltpu.SemaphoreType.DMA,
        ],
        compiler_params=pltpu.CompilerParams(use_tc_tiling_on_sc=False),
    )
    def sc_fn(idx_hbm, t0, t1, t2, t3, t4, out_hbm, idx_v, rows_v, sem0, sem1):
        wid = lax.axis_index("s") * 2 + lax.axis_index("c")
        tables = (t0, t1, t2, t3, t4)
        sems = (sem0, sem1)
        base = wid * per_w

        def load(g, slot):
            pltpu.sync_copy(idx_hbm.at[wid, g], idx_v.at[slot])
            for c in range(NTAB):
                pltpu.make_async_copy(
                    tables[c].at[idx_v.at[slot, c]],
                    rows_v.at[slot, c],
                    sems[slot],
                ).start()

        def process(g, slot):
            @pl.when(g + 1 < n_chunks)
            def _():
                load(g + 1, 1 - slot)

            for c in range(NTAB):
                pltpu.make_async_copy(
                    tables[c].at[idx_v.at[slot, c]],
                    rows_v.at[slot, c],
                    sems[slot],
                ).wait()

            def sum_row(r, carry):
                for cc in range(D // LANES):
                    sl = pl.ds(cc * LANES, LANES)
                    acc = (rows_v[slot, 0, r, sl]
                           + rows_v[slot, 1, r, sl]
                           + rows_v[slot, 2, r, sl]
                           + rows_v[slot, 3, r, sl]
                           + rows_v[slot, 4, r, sl])
                    rows_v[slot, 0, r, sl] = acc
                return carry

            lax.fori_loop(0, CHUNK, sum_row, 0)
            pltpu.sync_copy(rows_v.at[slot, 0],
                            out_hbm.at[pl.ds(base + g * CHUNK, CHUNK)])

        load(0, 0)

        def pair(p, carry):
            process(2 * p, 0)
            process(2 * p + 1, 1)
            return carry

        lax.fori_loop(0, n_chunks // 2, pair, 0)

    return sc_fn


def kernel(fonts, flag_w, contour_w, order_w, arg_w, fc_w, fc_b):
    b, s, en = fonts.shape
    n_tokens = b * s
    flagb, a0, a1 = _prep_tables(flag_w, arg_w, fc_w, fc_b)
    f = fonts.reshape(NW, n_tokens // (NW * CHUNK), CHUNK, en) + 1
    idx = f.transpose(0, 1, 3, 2)
    sc_fn = _make_sc_kernel(n_tokens)
    out = sc_fn(idx, a0, a1, flagb, contour_w, order_w)
    return out.reshape(b, s, D)
